# baseline (speedup 1.0000x reference)
import functools

import jax
import jax.numpy as jnp
from jax import lax
from jax.experimental import pallas as pl
from jax.experimental.pallas import tpu as pltpu
from jax.experimental.pallas import tpu_sc as plsc

_HIDDEN = 64
_BATCH = 16384
_HIST = 50
_HIST_PAD = 56
_HPAD = 128

_info = plsc.get_sparse_core_info()
_NC, _NS = _info.num_cores, _info.num_subcores
_NW = _NC * _NS
_A_PER_W = _BATCH // _NW
_ACHUNK = 128
_NACHUNK = _A_PER_W // _ACHUNK
_B_PER_W = _BATCH // _NW
_RING = 16
_DEPTH = 8
_NGROUP = _B_PER_W // _RING


def _make_gather():
    mesh = plsc.VectorSubcoreMesh(core_axis_name="c", subcore_axis_name="s")

    @functools.partial(
        pl.kernel,
        mesh=mesh,
        out_type=(
            jax.ShapeDtypeStruct((_BATCH, _HIDDEN), jnp.float32),
            jax.ShapeDtypeStruct((_BATCH, _HIST_PAD, _HPAD), jnp.float32),
        ),
        scratch_types=[
            pltpu.VMEM((_A_PER_W,), jnp.int32),
            pltpu.VMEM((_NACHUNK, _ACHUNK, _HIDDEN), jnp.float32),
            pltpu.VMEM((_B_PER_W, _HIST), jnp.int32),
            pltpu.VMEM((_RING, _HIST, _HIDDEN), jnp.float32),
            pltpu.SemaphoreType.DMA,
            pltpu.SemaphoreType.DMA,
        ],
        compiler_params=pltpu.CompilerParams(use_tc_tiling_on_sc=False),
    )
    def gather_kernel(ta_hbm, tb_hbm, table_hbm, outa_hbm, outb_hbm,
                      idx_a, arows, idx_b, brows, gsem, osem):
        wid = lax.axis_index("s") * _NC + lax.axis_index("c")
        abase = wid * _A_PER_W
        bbase = wid * _B_PER_W

        pltpu.sync_copy(ta_hbm.at[pl.ds(abase, _A_PER_W)], idx_a)
        for c in range(_NACHUNK):
            pltpu.async_copy(
                table_hbm.at[idx_a.at[pl.ds(c * _ACHUNK, _ACHUNK)]],
                arows.at[c], gsem)
        for c in range(_NACHUNK):
            pltpu.make_async_copy(
                table_hbm.at[idx_a.at[pl.ds(0, _ACHUNK)]], arows.at[c],
                gsem).wait()
            pltpu.async_copy(
                arows.at[c],
                outa_hbm.at[pl.ds(abase + c * _ACHUNK, _ACHUNK)], osem)
        for c in range(_NACHUNK):
            pltpu.make_async_copy(
                arows.at[0], outa_hbm.at[pl.ds(0, _ACHUNK)], osem).wait()

        pltpu.sync_copy(tb_hbm.at[pl.ds(bbase, _B_PER_W)], idx_b)

        def start_gather(r, slot):
            pltpu.async_copy(table_hbm.at[idx_b.at[r]], brows.at[slot],
                             gsem)

        for b in range(_DEPTH):
            start_gather(b, b)

        def body(g, carry):
            for b in range(_RING):
                r = g * _RING + b
                pltpu.make_async_copy(
                    table_hbm.at[idx_b.at[0]], brows.at[b], gsem).wait()
                pltpu.async_copy(
                    brows.at[b],
                    outb_hbm.at[bbase + r, pl.ds(0, _HIST),
                                pl.ds(0, _HIDDEN)], osem)
                @pl.when(r >= _RING - _DEPTH)
                def _():
                    pltpu.make_async_copy(
                        brows.at[0],
                        outb_hbm.at[0, pl.ds(0, _HIST), pl.ds(0, _HIDDEN)],
                        osem).wait()
                @pl.when(r + _DEPTH < _B_PER_W)
                def _():
                    start_gather(r + _DEPTH, (b + _DEPTH) % _RING)
            return carry

        lax.fori_loop(0, _NGROUP, body, 0)

        for _ in range(_RING - _DEPTH):
            pltpu.make_async_copy(
                brows.at[0],
                outb_hbm.at[0, pl.ds(0, _HIST), pl.ds(0, _HIDDEN)],
                osem).wait()

    return gather_kernel


_gather = _make_gather()

_VOCAB = 1000000
_DEPAD_R = 2000


def _depad(table):
    def body(in_ref, out_ref):
        x = in_ref[...]
        x3 = x.reshape(_DEPAD_R // 2, 2, _HIDDEN)
        out_ref[...] = jnp.concatenate([x3[:, 0, :], x3[:, 1, :]], axis=1)

    return pl.pallas_call(
        body,
        grid=(_VOCAB // _DEPAD_R,),
        in_specs=[pl.BlockSpec((_DEPAD_R, _HIDDEN), lambda i: (i, 0))],
        out_specs=pl.BlockSpec((_DEPAD_R // 2, 2 * _HIDDEN),
                               lambda i: (i, 0)),
        out_shape=jax.ShapeDtypeStruct((_VOCAB // 2, 2 * _HIDDEN),
                                       jnp.float32),
    )(table)


def kernel(tokens_a, tokens_b, embedding):
    lin = _depad(embedding).reshape(_VOCAB, _HIDDEN)
    out_a, out_b_pad = _gather(tokens_a.astype(jnp.int32),
                               tokens_b.astype(jnp.int32), lin)
    return (out_a, out_b_pad[:, :_HIST, :_HIDDEN])


# speedup vs baseline: 1.3990x; 1.3990x over previous
import functools

import jax
import jax.numpy as jnp
from jax import lax
from jax.experimental import pallas as pl
from jax.experimental.pallas import tpu as pltpu
from jax.experimental.pallas import tpu_sc as plsc

_HIDDEN = 64
_BATCH = 16384
_HIST = 50
_HIST_PAD = 56
_HPAD = 128

_info = plsc.get_sparse_core_info()
_NC, _NS = _info.num_cores, _info.num_subcores
_NW = _NC * _NS
_A_PER_W = _BATCH // _NW
_ACHUNK = 128
_NACHUNK = _A_PER_W // _ACHUNK
_B_PER_W = _BATCH // _NW
_RING = 16
_DEPTH = 8
_NGROUP = _B_PER_W // _RING


def _make_gather():
    mesh = plsc.VectorSubcoreMesh(core_axis_name="c", subcore_axis_name="s")

    @functools.partial(
        pl.kernel,
        mesh=mesh,
        out_type=(
            jax.ShapeDtypeStruct((_BATCH, _HIDDEN), jnp.float32),
            jax.ShapeDtypeStruct((_BATCH, _HIST_PAD, _HPAD), jnp.float32),
        ),
        scratch_types=[
            pltpu.VMEM((_A_PER_W,), jnp.int32),
            pltpu.VMEM((_NACHUNK, _ACHUNK, _HIDDEN), jnp.float32),
            pltpu.VMEM((_B_PER_W, _HIST), jnp.int32),
            pltpu.VMEM((_RING, _HIST, _HIDDEN), jnp.float32),
            pltpu.SemaphoreType.DMA,
            pltpu.SemaphoreType.DMA,
        ],
        compiler_params=pltpu.CompilerParams(use_tc_tiling_on_sc=False),
    )
    def gather_kernel(ta_hbm, tb_hbm, table_hbm, outa_hbm, outb_hbm,
                      idx_a, arows, idx_b, brows, gsem, osem):
        wid = lax.axis_index("s") * _NC + lax.axis_index("c")
        abase = wid * _A_PER_W
        bbase = wid * _B_PER_W

        pltpu.sync_copy(ta_hbm.at[pl.ds(abase, _A_PER_W)], idx_a)
        for c in range(_NACHUNK):
            pltpu.async_copy(
                table_hbm.at[idx_a.at[pl.ds(c * _ACHUNK, _ACHUNK)]],
                arows.at[c], gsem)
        for c in range(_NACHUNK):
            pltpu.make_async_copy(
                table_hbm.at[idx_a.at[pl.ds(0, _ACHUNK)]], arows.at[c],
                gsem).wait()
            pltpu.async_copy(
                arows.at[c],
                outa_hbm.at[pl.ds(abase + c * _ACHUNK, _ACHUNK)], osem)
        for c in range(_NACHUNK):
            pltpu.make_async_copy(
                arows.at[0], outa_hbm.at[pl.ds(0, _ACHUNK)], osem).wait()

        pltpu.sync_copy(tb_hbm.at[pl.ds(bbase, _B_PER_W)], idx_b)

        def start_gather(r, slot):
            pltpu.async_copy(table_hbm.at[idx_b.at[r]], brows.at[slot],
                             gsem)

        for b in range(_DEPTH):
            start_gather(b, b)

        def body(g, carry):
            for b in range(_RING):
                r = g * _RING + b
                pltpu.make_async_copy(
                    table_hbm.at[idx_b.at[0]], brows.at[b], gsem).wait()
                pltpu.async_copy(
                    brows.at[b],
                    outb_hbm.at[bbase + r, pl.ds(0, _HIST),
                                pl.ds(0, _HIDDEN)], osem)
                @pl.when(r >= _RING - _DEPTH)
                def _():
                    pltpu.make_async_copy(
                        brows.at[0],
                        outb_hbm.at[0, pl.ds(0, _HIST), pl.ds(0, _HIDDEN)],
                        osem).wait()
                @pl.when(r + _DEPTH < _B_PER_W)
                def _():
                    start_gather(r + _DEPTH, (b + _DEPTH) % _RING)
            return carry

        lax.fori_loop(0, _NGROUP, body, 0)

        for _ in range(_RING - _DEPTH):
            pltpu.make_async_copy(
                brows.at[0],
                outb_hbm.at[0, pl.ds(0, _HIST), pl.ds(0, _HIDDEN)],
                osem).wait()

    return gather_kernel


_gather = _make_gather()

_VOCAB = 1000000
_DEPAD_C = 2048


def _tdepad(table_t):
    def body(in_ref, out_ref):
        xt = in_ref[...].T
        x3 = xt.reshape(_DEPAD_C // 2, 2, _HIDDEN)
        out_ref[...] = jnp.concatenate([x3[:, 0, :], x3[:, 1, :]], axis=1)

    return pl.pallas_call(
        body,
        grid=(pl.cdiv(_VOCAB, _DEPAD_C),),
        in_specs=[pl.BlockSpec((_HIDDEN, _DEPAD_C), lambda i: (0, i))],
        out_specs=pl.BlockSpec((_DEPAD_C // 2, 2 * _HIDDEN),
                               lambda i: (i, 0)),
        out_shape=jax.ShapeDtypeStruct((_VOCAB // 2, 2 * _HIDDEN),
                                       jnp.float32),
    )(table_t)


def kernel(tokens_a, tokens_b, embedding):
    lin = _tdepad(embedding.T).reshape(_VOCAB, _HIDDEN)
    out_a, out_b_pad = _gather(tokens_a.astype(jnp.int32),
                               tokens_b.astype(jnp.int32), lin)
    return (out_a, out_b_pad[:, :_HIST, :_HIDDEN])


# speedup vs baseline: 1.6478x; 1.1779x over previous
import functools

import jax
import jax.numpy as jnp
from jax import lax
from jax.experimental import pallas as pl
from jax.experimental.pallas import tpu as pltpu
from jax.experimental.pallas import tpu_sc as plsc

_HIDDEN = 64
_BATCH = 16384
_HIST = 50
_HIST_PAD = 56
_HPAD = 128

_info = plsc.get_sparse_core_info()
_NC, _NS = _info.num_cores, _info.num_subcores
_NW = _NC * _NS
_A_PER_W = _BATCH // _NW
_ACHUNK = 128
_NACHUNK = _A_PER_W // _ACHUNK
_B_PER_W = _BATCH // _NW
_RING = 16
_DEPTH = 8
_NGROUP = _B_PER_W // _RING


def _make_gather():
    mesh = plsc.VectorSubcoreMesh(core_axis_name="c", subcore_axis_name="s")

    @functools.partial(
        pl.kernel,
        mesh=mesh,
        out_type=(
            jax.ShapeDtypeStruct((_BATCH, _HIDDEN), jnp.float32),
            jax.ShapeDtypeStruct((_BATCH, _HIST_PAD, _HPAD), jnp.float32),
        ),
        scratch_types=[
            pltpu.VMEM((_A_PER_W,), jnp.int32),
            pltpu.VMEM((_NACHUNK, _ACHUNK, _HIDDEN), jnp.float32),
            pltpu.VMEM((_B_PER_W, _HIST), jnp.int32),
            pltpu.VMEM((_RING, _HIST, _HIDDEN), jnp.float32),
            pltpu.SemaphoreType.DMA,
            pltpu.SemaphoreType.DMA,
        ],
        compiler_params=pltpu.CompilerParams(use_tc_tiling_on_sc=False),
    )
    def gather_kernel(ta_hbm, tb_hbm, table_hbm, outa_hbm, outb_hbm,
                      idx_a, arows, idx_b, brows, gsem, osem):
        wid = lax.axis_index("s") * _NC + lax.axis_index("c")
        abase = wid * _A_PER_W
        bbase = wid * _B_PER_W

        pltpu.sync_copy(ta_hbm.at[pl.ds(abase, _A_PER_W)], idx_a)
        for c in range(_NACHUNK):
            pltpu.async_copy(
                table_hbm.at[idx_a.at[pl.ds(c * _ACHUNK, _ACHUNK)]],
                arows.at[c], gsem)
        for c in range(_NACHUNK):
            pltpu.make_async_copy(
                table_hbm.at[idx_a.at[pl.ds(0, _ACHUNK)]], arows.at[c],
                gsem).wait()
            pltpu.async_copy(
                arows.at[c],
                outa_hbm.at[pl.ds(abase + c * _ACHUNK, _ACHUNK)], osem)
        for c in range(_NACHUNK):
            pltpu.make_async_copy(
                arows.at[0], outa_hbm.at[pl.ds(0, _ACHUNK)], osem).wait()

        pltpu.sync_copy(tb_hbm.at[pl.ds(bbase, _B_PER_W)], idx_b)

        def start_gather(r, slot):
            pltpu.async_copy(table_hbm.at[idx_b.at[r]], brows.at[slot],
                             gsem)

        for b in range(_DEPTH):
            start_gather(b, b)

        def body(g, carry):
            for b in range(_RING):
                r = g * _RING + b
                pltpu.make_async_copy(
                    table_hbm.at[idx_b.at[0]], brows.at[b], gsem).wait()
                pltpu.async_copy(
                    brows.at[b],
                    outb_hbm.at[bbase + r, pl.ds(0, _HIST),
                                pl.ds(0, _HIDDEN)], osem)
                @pl.when(r >= _RING - _DEPTH)
                def _():
                    pltpu.make_async_copy(
                        brows.at[0],
                        outb_hbm.at[0, pl.ds(0, _HIST), pl.ds(0, _HIDDEN)],
                        osem).wait()
                @pl.when(r + _DEPTH < _B_PER_W)
                def _():
                    start_gather(r + _DEPTH, (b + _DEPTH) % _RING)
            return carry

        lax.fori_loop(0, _NGROUP, body, 0)

        for _ in range(_RING - _DEPTH):
            pltpu.make_async_copy(
                brows.at[0],
                outb_hbm.at[0, pl.ds(0, _HIST), pl.ds(0, _HIDDEN)],
                osem).wait()

    return gather_kernel


_gather = _make_gather()

_VOCAB = 1000000
_DEPAD_C = 8192


def _tdepad(table_t):
    def body(in_ref, out_ref):
        xt = in_ref[...].T
        x3 = xt.reshape(_DEPAD_C // 2, 2, _HIDDEN)
        out_ref[...] = jnp.concatenate([x3[:, 0, :], x3[:, 1, :]], axis=1)

    return pl.pallas_call(
        body,
        grid=(pl.cdiv(_VOCAB, _DEPAD_C),),
        in_specs=[pl.BlockSpec((_HIDDEN, _DEPAD_C), lambda i: (0, i))],
        out_specs=pl.BlockSpec((_DEPAD_C // 2, 2 * _HIDDEN),
                               lambda i: (i, 0)),
        out_shape=jax.ShapeDtypeStruct((_VOCAB // 2, 2 * _HIDDEN),
                                       jnp.float32),
    )(table_t)


def kernel(tokens_a, tokens_b, embedding):
    lin = _tdepad(embedding.T).reshape(_VOCAB, _HIDDEN)
    out_a, out_b_pad = _gather(tokens_a.astype(jnp.int32),
                               tokens_b.astype(jnp.int32), lin)
    return (out_a, out_b_pad[:, :_HIST, :_HIDDEN])


# speedup vs baseline: 1.6532x; 1.0032x over previous
import functools

import jax
import jax.numpy as jnp
from jax import lax
from jax.experimental import pallas as pl
from jax.experimental.pallas import tpu as pltpu
from jax.experimental.pallas import tpu_sc as plsc

_HIDDEN = 64
_BATCH = 16384
_HIST = 50
_HIST_PAD = 56
_HPAD = 128

_info = plsc.get_sparse_core_info()
_NC, _NS = _info.num_cores, _info.num_subcores
_NW = _NC * _NS
_A_PER_W = _BATCH // _NW
_ACHUNK = 128
_NACHUNK = _A_PER_W // _ACHUNK
_B_PER_W = _BATCH // _NW
_RING = 16
_DEPTH = 8
_NGROUP = _B_PER_W // _RING


def _make_gather():
    mesh = plsc.VectorSubcoreMesh(core_axis_name="c", subcore_axis_name="s")

    @functools.partial(
        pl.kernel,
        mesh=mesh,
        out_type=(
            jax.ShapeDtypeStruct((_BATCH, _HIDDEN), jnp.float32),
            jax.ShapeDtypeStruct((_BATCH, _HIST_PAD, _HPAD), jnp.float32),
        ),
        scratch_types=[
            pltpu.VMEM((_A_PER_W,), jnp.int32),
            pltpu.VMEM((_NACHUNK, _ACHUNK, _HIDDEN), jnp.float32),
            pltpu.VMEM((_B_PER_W, _HIST), jnp.int32),
            pltpu.VMEM((_RING, _HIST, _HIDDEN), jnp.float32),
            pltpu.SemaphoreType.DMA,
            pltpu.SemaphoreType.DMA,
        ],
        compiler_params=pltpu.CompilerParams(use_tc_tiling_on_sc=False),
    )
    def gather_kernel(ta_hbm, tb_hbm, table_hbm, outa_hbm, outb_hbm,
                      idx_a, arows, idx_b, brows, gsem, osem):
        wid = lax.axis_index("s") * _NC + lax.axis_index("c")
        abase = wid * _A_PER_W
        bbase = wid * _B_PER_W

        pltpu.sync_copy(ta_hbm.at[pl.ds(abase, _A_PER_W)], idx_a)
        for c in range(_NACHUNK):
            pltpu.async_copy(
                table_hbm.at[idx_a.at[pl.ds(c * _ACHUNK, _ACHUNK)]],
                arows.at[c], gsem)
        for c in range(_NACHUNK):
            pltpu.make_async_copy(
                table_hbm.at[idx_a.at[pl.ds(0, _ACHUNK)]], arows.at[c],
                gsem).wait()
            pltpu.async_copy(
                arows.at[c],
                outa_hbm.at[pl.ds(abase + c * _ACHUNK, _ACHUNK)], osem)
        for c in range(_NACHUNK):
            pltpu.make_async_copy(
                arows.at[0], outa_hbm.at[pl.ds(0, _ACHUNK)], osem).wait()

        pltpu.sync_copy(tb_hbm.at[pl.ds(bbase, _B_PER_W)], idx_b)

        def start_gather(r, slot):
            pltpu.async_copy(table_hbm.at[idx_b.at[r]], brows.at[slot],
                             gsem)

        for b in range(_DEPTH):
            start_gather(b, b)

        def body(g, carry):
            for b in range(_RING):
                r = g * _RING + b
                pltpu.make_async_copy(
                    table_hbm.at[idx_b.at[0]], brows.at[b], gsem).wait()
                pltpu.async_copy(
                    brows.at[b],
                    outb_hbm.at[bbase + r, pl.ds(0, _HIST),
                                pl.ds(0, _HIDDEN)], osem)
                @pl.when(r >= _RING - _DEPTH)
                def _():
                    pltpu.make_async_copy(
                        brows.at[0],
                        outb_hbm.at[0, pl.ds(0, _HIST), pl.ds(0, _HIDDEN)],
                        osem).wait()
                @pl.when(r + _DEPTH < _B_PER_W)
                def _():
                    start_gather(r + _DEPTH, (b + _DEPTH) % _RING)
            return carry

        lax.fori_loop(0, _NGROUP, body, 0)

        for _ in range(_RING - _DEPTH):
            pltpu.make_async_copy(
                brows.at[0],
                outb_hbm.at[0, pl.ds(0, _HIST), pl.ds(0, _HIDDEN)],
                osem).wait()

    return gather_kernel


_gather = _make_gather()

_VOCAB = 1000000
_DEPAD_C = 16384


def _tdepad(table_t):
    def body(in_ref, out_ref):
        xt = in_ref[...].T
        x3 = xt.reshape(_DEPAD_C // 2, 2, _HIDDEN)
        out_ref[...] = jnp.concatenate([x3[:, 0, :], x3[:, 1, :]], axis=1)

    return pl.pallas_call(
        body,
        grid=(pl.cdiv(_VOCAB, _DEPAD_C),),
        in_specs=[pl.BlockSpec((_HIDDEN, _DEPAD_C), lambda i: (0, i))],
        out_specs=pl.BlockSpec((_DEPAD_C // 2, 2 * _HIDDEN),
                               lambda i: (i, 0)),
        out_shape=jax.ShapeDtypeStruct((_VOCAB // 2, 2 * _HIDDEN),
                                       jnp.float32),
    )(table_t)


def kernel(tokens_a, tokens_b, embedding):
    lin = _tdepad(embedding.T).reshape(_VOCAB, _HIDDEN)
    out_a, out_b_pad = _gather(tokens_a.astype(jnp.int32),
                               tokens_b.astype(jnp.int32), lin)
    return (out_a, out_b_pad[:, :_HIST, :_HIDDEN])
